# Initial kernel scaffold; baseline (speedup 1.0000x reference)
import jax
import jax.numpy as jnp
from jax.experimental import pallas as pl


def kernel(cloth_properties, external_forces, gaussian_positions, gaussian_scales, gaussian_rotations, gaussian_opacities, gaussian_features, num_steps):
    raise NotImplementedError("write your pallas kernel here")


# speedup vs baseline: 1.0306x; 1.0306x over previous
import functools

import jax
import jax.numpy as jnp
from jax import lax
from jax.experimental import pallas as pl
from jax.experimental.pallas import tpu as pltpu
from jax.experimental.pallas import tpu_sc as plsc

N = 10000
L = 16
NW = 32
CHUNK = 320
PADN = NW * CHUNK
FRONT = 16
WIN = FRONT + CHUNK + 16
PLEN = FRONT + PADN + 16
G = CHUNK // L
DT = 0.016
REST = 0.05
MAXD = 9


def _rsqrt(x):
    i = lax.bitcast_convert_type(x, jnp.int32)
    i = jnp.int32(0x5F3759DF) - (i >> 1)
    y = lax.bitcast_convert_type(i, jnp.float32)
    xh = x * 0.5
    for _ in range(3):
        y = y * (1.5 - xh * y * y)
    return y


_mesh = plsc.VectorSubcoreMesh(core_axis_name="c", subcore_axis_name="s")


@functools.partial(
    pl.kernel,
    out_type=jax.ShapeDtypeStruct((6, PADN), jnp.float32),
    mesh=_mesh,
    scratch_types=[
        pltpu.VMEM((4, WIN), jnp.float32),
        pltpu.VMEM((8, CHUNK), jnp.float32),
        pltpu.VMEM((6, CHUNK), jnp.float32),
    ],
    compiler_params=pltpu.CompilerParams(use_tc_tiling_on_sc=False),
)
def _step_kernel(posk_hbm, attr_hbm, out_hbm, posk_v, attr_v, out_v):
    c = lax.axis_index("c")
    s = lax.axis_index("s")
    wid = s * 2 + c
    base = wid * CHUNK

    pltpu.sync_copy(posk_hbm.at[:, pl.ds(base, WIN)], posk_v)
    pltpu.sync_copy(attr_hbm.at[:, pl.ds(base, CHUNK)], attr_v)

    iota = lax.iota(jnp.int32, L)

    def body(g, carry):
        lo = FRONT + g * L
        og = g * L
        px = posk_v[0, pl.ds(lo, L)]
        py = posk_v[1, pl.ds(lo, L)]
        pz = posk_v[2, pl.ds(lo, L)]
        ki = posk_v[3, pl.ds(lo, L)]
        r = base + og + iota
        fx = attr_v[0, pl.ds(og, L)]
        fy = attr_v[1, pl.ds(og, L)] + (-9.81)
        fz = attr_v[2, pl.ds(og, L)]
        for d in range(1, MAXD + 1):
            dx = posk_v[0, pl.ds(lo + d, L)] - px
            dy = posk_v[1, pl.ds(lo + d, L)] - py
            dz = posk_v[2, pl.ds(lo + d, L)] - pz
            d2 = dx * dx + dy * dy + dz * dz
            cf = ki * (1.0 - REST * _rsqrt(d2))
            cf = jnp.where((r < N - d) & (d2 > 0.0), cf, 0.0)
            fx = fx + cf * dx
            fy = fy + cf * dy
            fz = fz + cf * dz
            ex = px - posk_v[0, pl.ds(lo - d, L)]
            ey = py - posk_v[1, pl.ds(lo - d, L)]
            ez = pz - posk_v[2, pl.ds(lo - d, L)]
            e2 = ex * ex + ey * ey + ez * ez
            km = posk_v[3, pl.ds(lo - d, L)]
            cm = km * (1.0 - REST * _rsqrt(e2))
            cm = jnp.where((r >= d) & (e2 > 0.0), cm, 0.0)
            fx = fx - cm * ex
            fy = fy - cm * ey
            fz = fz - cm * ez
        fy = fy + jnp.where(py < -1.0, 1000.0 * (-1.0 - py), 0.0)
        vx = attr_v[3, pl.ds(og, L)]
        vy = attr_v[4, pl.ds(og, L)]
        vz = attr_v[5, pl.ds(og, L)]
        inv = 1.0 / (attr_v[6, pl.ds(og, L)] + 1e-8)
        ax = fx * inv
        ay = fy * inv
        az = fz * inv
        half = 0.5 * DT * DT
        out_v[0, pl.ds(og, L)] = px + vx * DT + ax * half
        out_v[1, pl.ds(og, L)] = py + vy * DT + ay * half
        out_v[2, pl.ds(og, L)] = pz + vz * DT + az * half
        dfac = 1.0 - attr_v[7, pl.ds(og, L)] * DT
        out_v[3, pl.ds(og, L)] = (vx + ax * DT) * dfac
        out_v[4, pl.ds(og, L)] = (vy + ay * DT) * dfac
        out_v[5, pl.ds(og, L)] = (vz + az * DT) * dfac
        return carry

    lax.fori_loop(0, G, body, jnp.int32(0))
    pltpu.sync_copy(out_v, out_hbm.at[:, pl.ds(base, CHUNK)])


def kernel(cloth_properties, external_forces, gaussian_positions,
           gaussian_scales, gaussian_rotations, gaussian_opacities,
           gaussian_features, num_steps):
    stiff = cloth_properties[:, 0]
    damping = cloth_properties[:, 1]
    mass = cloth_properties[:, 6]
    ext = external_forces.T
    md = jnp.stack([mass, damping], axis=0)

    def step(carry):
        pos, vel = carry
        posk = jnp.concatenate([pos.T, stiff[None, :]], axis=0)
        posk = jnp.pad(posk, ((0, 0), (FRONT, PLEN - FRONT - N)))
        attr = jnp.concatenate([ext, vel.T, md], axis=0)
        attr = jnp.pad(attr, ((0, 0), (0, PADN - N)))
        out = _step_kernel(posk, attr)
        return out[:3, :N].T, out[3:, :N].T

    pos0 = gaussian_positions
    vel0 = jnp.zeros_like(pos0)
    pos, vel = lax.fori_loop(0, num_steps, lambda i, cr: step(cr),
                             (pos0, vel0))
    return (pos, vel, gaussian_scales, gaussian_rotations,
            gaussian_opacities, gaussian_features)
